# baseline (speedup 1.0000x reference)
import functools

import jax
import jax.numpy as jnp
from jax import lax
from jax.experimental import pallas as pl
from jax.experimental.pallas import tpu as pltpu
from jax.experimental.pallas import tpu_sc as plsc

B = 16384
D = 32
NC = 2
NS = 16
L = 16
NW = NC * NS
BPW = B // NW
CHUNK = 128
NCH = BPW // CHUNK
NTILES = BPW // L

_mesh = plsc.VectorSubcoreMesh(core_axis_name="c", subcore_axis_name="s")


@functools.partial(
    pl.kernel,
    out_type=jax.ShapeDtypeStruct((B,), jnp.float32),
    mesh=_mesh,
    compiler_params=pltpu.CompilerParams(
        needs_layout_passes=False, use_tc_tiling_on_sc=False),
    scratch_types=[
        pltpu.VMEM((NCH, CHUNK), jnp.int32),
        pltpu.VMEM((NCH, CHUNK), jnp.int32),
        pltpu.VMEM((BPW, D), jnp.float32),
        pltpu.VMEM((BPW, D), jnp.float32),
        pltpu.VMEM((80,), jnp.float32),
        pltpu.VMEM((BPW,), jnp.float32),
        pltpu.SemaphoreType.DMA,
    ],
)
def _sc_forward(users_hbm, movies_hbm, ut_hbm, mt_hbm, wb_hbm, out_hbm,
                uidx_v, midx_v, urows_v, mrows_v, wb_v, out_v, sem):
    wid = lax.axis_index("s") * NC + lax.axis_index("c")
    pltpu.sync_copy(users_hbm.at[pl.ds(wid * NCH, NCH)], uidx_v)
    pltpu.sync_copy(movies_hbm.at[pl.ds(wid * NCH, NCH)], midx_v)
    pltpu.sync_copy(wb_hbm, wb_v)
    copies = []
    for j in range(NCH):
        copies.append(pltpu.async_copy(
            ut_hbm.at[uidx_v.at[j]], urows_v.at[pl.ds(j * CHUNK, CHUNK)], sem))
        copies.append(pltpu.async_copy(
            mt_hbm.at[midx_v.at[j]], mrows_v.at[pl.ds(j * CHUNK, CHUNK)], sem))
    for c in copies:
        c.wait()

    lane = lax.iota(jnp.int32, L)
    wvec = [wb_v[pl.ds(k * L, L)] for k in range(4)]
    bias = wb_v[pl.ds(64, L)][0]

    def tile_body(t, _):
        row_idx = t * L + lane
        acc = jnp.full((L,), bias, jnp.float32)
        for d in range(D):
            col = jnp.full((L,), d, jnp.int32)
            cu = plsc.load_gather(urows_v, [row_idx, col])
            acc = acc + cu * wvec[d // L][d % L]
            cm = plsc.load_gather(mrows_v, [row_idx, col])
            acc = acc + cm * wvec[2 + d // L][d % L]
        out_v[pl.ds(t * L, L)] = acc
        return 0

    lax.fori_loop(0, NTILES, tile_body, 0)
    pltpu.sync_copy(out_v, out_hbm.at[pl.ds(wid * BPW, BPW)])


def kernel(users, movies, user_table, movie_table, W, b):
    wb = jnp.concatenate(
        [W.reshape(-1).astype(jnp.float32), b.astype(jnp.float32),
         jnp.zeros((15,), jnp.float32)])
    u2 = users.astype(jnp.int32).reshape(NW * NCH, CHUNK)
    m2 = movies.astype(jnp.int32).reshape(NW * NCH, CHUNK)
    out = _sc_forward(u2, m2, user_table, movie_table, wb)
    return out.reshape(B, 1)


# speedup vs baseline: 6.7833x; 6.7833x over previous
import functools

import jax
import jax.numpy as jnp
from jax import lax
from jax.experimental import pallas as pl
from jax.experimental.pallas import tpu as pltpu
from jax.experimental.pallas import tpu_sc as plsc

B = 16384
D = 32
NC = 2
NS = 16
L = 16
NW = NC * NS
BPW = B // NW
CHUNK = 128
NCH = BPW // CHUNK

BC = 32768


def _mv_body(x_ref, w_ref, b_ref, o_ref):
    o_ref[...] = jnp.sum(x_ref[...] * w_ref[...], axis=0) + b_ref[0, 0]


def _matvec(table_t, w, bias):
    n = table_t.shape[1]
    grid = pl.cdiv(n, BC)
    return pl.pallas_call(
        _mv_body,
        grid=(grid,),
        in_specs=[
            pl.BlockSpec((D, BC), lambda i: (0, i)),
            pl.BlockSpec((D, 1), lambda i: (0, 0)),
            pl.BlockSpec((1, 1), lambda i: (0, 0)),
        ],
        out_specs=pl.BlockSpec((BC,), lambda i: (i,)),
        out_shape=jax.ShapeDtypeStruct((n,), jnp.float32),
    )(table_t, w, bias)


_mesh = plsc.VectorSubcoreMesh(core_axis_name="c", subcore_axis_name="s")


@functools.partial(
    pl.kernel,
    out_type=jax.ShapeDtypeStruct((B,), jnp.float32),
    mesh=_mesh,
    compiler_params=pltpu.CompilerParams(
        needs_layout_passes=False, use_tc_tiling_on_sc=False),
    scratch_types=[
        pltpu.VMEM((NCH, CHUNK), jnp.int32),
        pltpu.VMEM((NCH, CHUNK), jnp.int32),
        pltpu.VMEM((BPW,), jnp.float32),
        pltpu.VMEM((BPW,), jnp.float32),
        pltpu.VMEM((BPW,), jnp.float32),
        pltpu.SemaphoreType.DMA,
    ],
)
def _sc_gather_add(users_hbm, movies_hbm, su_hbm, sm_hbm, out_hbm,
                   uidx_v, midx_v, su_v, sm_v, out_v, sem):
    wid = lax.axis_index("s") * NC + lax.axis_index("c")
    pltpu.sync_copy(users_hbm.at[pl.ds(wid * NCH, NCH)], uidx_v)
    pltpu.sync_copy(movies_hbm.at[pl.ds(wid * NCH, NCH)], midx_v)
    copies = []
    for j in range(NCH):
        copies.append(pltpu.async_copy(
            su_hbm.at[uidx_v.at[j]], su_v.at[pl.ds(j * CHUNK, CHUNK)], sem))
        copies.append(pltpu.async_copy(
            sm_hbm.at[midx_v.at[j]], sm_v.at[pl.ds(j * CHUNK, CHUNK)], sem))
    for c in copies:
        c.wait()
    for i in range(BPW // L):
        sl = pl.ds(i * L, L)
        out_v[sl] = su_v[sl] + sm_v[sl]
    pltpu.sync_copy(out_v, out_hbm.at[pl.ds(wid * BPW, BPW)])


def kernel(users, movies, user_table, movie_table, W, b):
    wf = W.reshape(-1).astype(jnp.float32)
    wu = wf[:D].reshape(D, 1)
    wm = wf[D:].reshape(D, 1)
    bias = b.astype(jnp.float32).reshape(1, 1)
    zero = jnp.zeros((1, 1), jnp.float32)
    su = _matvec(user_table.T, wu, bias)
    sm = _matvec(movie_table.T, wm, zero)
    u2 = users.astype(jnp.int32).reshape(NW * NCH, CHUNK)
    m2 = movies.astype(jnp.int32).reshape(NW * NCH, CHUNK)
    out = _sc_gather_add(u2, m2, su, sm)
    return out.reshape(B, 1)
